# baseline (speedup 1.0000x reference)
import jax
import jax.numpy as jnp
from jax import lax
from jax.experimental import pallas as pl
from jax.experimental.pallas import tpu as pltpu
from jax.experimental.pallas import tpu_sc as plsc

N = 8192
D = 256
ALPHA = 0.2
RB_H = 512
RB = 128
N_SC = 128
NW = 32
G = 32
L = 16


def _h_kernel(x_ref, w_ref, asrc_ref, adst_ref, hbf_ref, hf_ref,
              av_src_ref, av_dst_ref):
    h = jax.lax.dot_general(
        x_ref[...], w_ref[...],
        dimension_numbers=(((1,), (1,)), ((), ())),
        preferred_element_type=jnp.float32,
    )
    hf_ref[...] = h
    hbf_ref[...] = h.astype(jnp.bfloat16)
    av_src_ref[...] = jnp.sum(h * asrc_ref[...], axis=1, keepdims=True)
    av_dst_ref[...] = jnp.sum(h * adst_ref[...], axis=1, keepdims=True)


def _attn_kernel(adj_ref, asrc_row_ref, adst_col_ref, hbf_ref, out_ref):
    s = asrc_row_ref[...] + adst_col_ref[...]
    s = jnp.where(s > 0, s, ALPHA * s)
    mask = adj_ref[...] > 0
    s = jnp.where(mask, s, 0.0)
    rowmax = jnp.max(s, axis=1, keepdims=True)
    e = jnp.where(mask, jnp.exp(s - rowmax), 0.0)
    inv = 1.0 / (jnp.sum(e, axis=1, keepdims=True) + 1e-8)
    out = jnp.dot(e.astype(jnp.bfloat16), hbf_ref[...],
                  preferred_element_type=jnp.float32) * inv
    out_ref[...] = jnp.where(out > 0, out, jnp.exp(jnp.minimum(out, 0.0)) - 1.0)


def _sc_attn(adj_hbm, asrc_hbm, adst_hbm, h_hbm, out_hbm,
             rowbuf, asrc_v, adst_v, idx_v, w_v, idxc_v, hrows_v, orow_v,
             smem_i, smem_f, sem):
    rows_per_w = N_SC // NW
    wid = lax.axis_index("s") * 2 + lax.axis_index("c")
    row0 = wid * rows_per_w
    zeros = jnp.zeros((L,), jnp.float32)
    zeros_i = jnp.zeros((L,), jnp.int32)
    lane = lax.iota(jnp.int32, L)

    pltpu.sync_copy(asrc_hbm, asrc_v)
    pltpu.sync_copy(adst_hbm, adst_v)

    def row_body(r, carry):
        i = row0 + r
        pltpu.sync_copy(adj_hbm.at[i], rowbuf)

        cb = (i // L) * L
        chunk_d = adst_v[pl.ds(cb, L)]
        sel = jnp.where(lane == (i - cb), chunk_d, 0.0)
        smem_f[2] = jnp.sum(sel)
        adsti = smem_f[2]

        def scan_body(gidx, carry1):
            base = gidx * 4 * L
            v0 = rowbuf[pl.ds(base, L)]
            v1 = rowbuf[pl.ds(base + L, L)]
            v2 = rowbuf[pl.ds(base + 2 * L, L)]
            v3 = rowbuf[pl.ds(base + 3 * L, L)]
            t = jnp.sum(v0 + v1 + v2 + v3)

            def nonempty(c1):
                nnz_c, mrow_c = c1
                for k, vk in enumerate((v0, v1, v2, v3)):
                    m = vk > 0.0
                    mi = m.astype(jnp.int32)
                    s = asrc_v[pl.ds(base + k * L, L)] + adsti
                    s = jnp.where(s > 0, s, ALPHA * s)
                    sm = jnp.where(m, s, -3e38)
                    mrow_c = jnp.maximum(mrow_c, jnp.max(sm))
                    pos = nnz_c + plsc.cumsum(mi) - 1
                    idxc_v[pl.ds(0, L)] = pos
                    pos_rt = idxc_v[pl.ds(0, L)]
                    plsc.store_scatter(idx_v, [pos_rt],
                                       lane + base + k * L, mask=m)
                    plsc.store_scatter(w_v, [pos_rt], s, mask=m)
                    nnz_c = nnz_c + jnp.sum(mi)
                smem_i[0] = nnz_c
                smem_f[0] = mrow_c
                return (nnz_c, mrow_c)

            return lax.cond(t > 0.0, nonempty, lambda c1: c1, carry1)
        smem_i[0] = 0
        smem_f[0] = -3e38
        lax.fori_loop(0, N // (4 * L), scan_body, (0, -3e38))
        nnz = smem_i[0]
        m_row = smem_f[0]
        m_row = jnp.where(nnz < N, jnp.maximum(m_row, 0.0), m_row)

        for k in range(G // L + 1):
            idx_v[pl.ds(nnz + k * L, L)] = zeros_i

        for k in range(G // L):
            idxc_v[pl.ds(k * L, L)] = idx_v[pl.ds(k * L, L)]
        cp0 = pltpu.async_copy(h_hbm.at[idxc_v], hrows_v, sem)

        eb = (nnz + L - 1) // L
        smem_f[1] = 0.0

        def exp_body(k, sum_c):
            base = k * L
            e = jnp.exp(w_v[pl.ds(base, L)] - m_row)
            e = jnp.where((lane + base) < nnz, e, 0.0)
            w_v[pl.ds(base, L)] = e
            sum_n = sum_c + jnp.sum(e)
            smem_f[1] = sum_n
            return sum_n
        lax.fori_loop(0, eb, exp_body, 0.0)
        inv = 1.0 / (jnp.full((L,), smem_f[1], jnp.float32) + 1e-8)
        for k in range(G // L + 1):
            w_v[pl.ds(nnz + k * L, L)] = zeros

        for c in range(D // L):
            orow_v[pl.ds(c * L, L)] = zeros

        def accumulate(base):
            for sub in range(G // L):
                wchunk = w_v[pl.ds(base + sub * L, L)]
                for e in range(L):
                    wb = zeros + wchunk[e]
                    row = sub * L + e
                    for c in range(D // L):
                        plsc.addupdate(orow_v.at[pl.ds(c * L, L)],
                                       wb * hrows_v[row, pl.ds(c * L, L)])

        nch = (nnz + G - 1) // G
        cp0.wait()
        accumulate(0)

        def chunk_body(g, carry2):
            base = g * G
            for k in range(G // L):
                idxc_v[pl.ds(k * L, L)] = idx_v[pl.ds(base + k * L, L)]
            pltpu.async_copy(h_hbm.at[idxc_v], hrows_v, sem).wait()
            accumulate(base)
            return carry2
        lax.fori_loop(1, nch, chunk_body, 0)

        for c in range(D // L):
            v = orow_v[pl.ds(c * L, L)] * inv
            orow_v[pl.ds(c * L, L)] = jnp.where(
                v > 0, v, jnp.exp(jnp.minimum(v, 0.0)) - 1.0)
        pltpu.sync_copy(orow_v, out_hbm.at[i])
        return carry

    lax.fori_loop(0, rows_per_w, row_body, 0)


def kernel(x, adj, W, attn_src, attn_dst):
    hbf, hf, a_src, a_dst = pl.pallas_call(
        _h_kernel,
        grid=(N // RB_H,),
        in_specs=[
            pl.BlockSpec((RB_H, D), lambda i: (i, 0)),
            pl.BlockSpec((D, D), lambda i: (0, 0)),
            pl.BlockSpec((1, D), lambda i: (0, 0)),
            pl.BlockSpec((1, D), lambda i: (0, 0)),
        ],
        out_specs=[
            pl.BlockSpec((RB_H, D), lambda i: (i, 0)),
            pl.BlockSpec((RB_H, D), lambda i: (i, 0)),
            pl.BlockSpec((RB_H, 1), lambda i: (i, 0)),
            pl.BlockSpec((RB_H, 1), lambda i: (i, 0)),
        ],
        out_shape=[
            jax.ShapeDtypeStruct((N, D), jnp.bfloat16),
            jax.ShapeDtypeStruct((N, D), jnp.float32),
            jax.ShapeDtypeStruct((N, 1), jnp.float32),
            jax.ShapeDtypeStruct((N, 1), jnp.float32),
        ],
    )(x, W, attn_src, attn_dst)

    parts = []
    if N_SC < N:
        a_src_row = a_src.reshape(1, N)
        out_tc = pl.pallas_call(
            _attn_kernel,
            grid=((N - N_SC) // RB,),
            in_specs=[
                pl.BlockSpec((RB, N), lambda i: (i + N_SC // RB, 0)),
                pl.BlockSpec((1, N), lambda i: (0, 0)),
                pl.BlockSpec((RB, 1), lambda i: (i + N_SC // RB, 0)),
                pl.BlockSpec((N, D), lambda i: (0, 0)),
            ],
            out_specs=pl.BlockSpec((RB, D), lambda i: (i, 0)),
            out_shape=jax.ShapeDtypeStruct((N - N_SC, D), jnp.float32),
        )(adj, a_src_row, a_dst, hbf)
        parts.append(out_tc)

    if N_SC > 0:
        mesh = plsc.VectorSubcoreMesh(core_axis_name="c", subcore_axis_name="s")
        sc_fn = pl.kernel(
            _sc_attn, mesh=mesh,
            compiler_params=pltpu.CompilerParams(needs_layout_passes=False),
            out_type=jax.ShapeDtypeStruct((N_SC, D), jnp.float32),
            scratch_types=[
                pltpu.VMEM((N,), jnp.float32),
                pltpu.VMEM((N,), jnp.float32),
                pltpu.VMEM((N,), jnp.float32),
                pltpu.VMEM((N + 4 * L,), jnp.int32),
                pltpu.VMEM((N + 4 * L,), jnp.float32),
                pltpu.VMEM((G,), jnp.int32),
                pltpu.VMEM((G, D), jnp.float32),
                pltpu.VMEM((D,), jnp.float32),
                pltpu.SMEM((4,), jnp.int32),
                pltpu.SMEM((4,), jnp.float32),
                pltpu.SemaphoreType.DMA,
            ],
        )
        out_sc = sc_fn(adj, a_src.reshape(N), a_dst.reshape(N), hf)
        parts.insert(0, out_sc)

    return parts[0] if len(parts) == 1 else jnp.concatenate(parts, axis=0)


# speedup vs baseline: 1.0307x; 1.0307x over previous
import jax
import jax.numpy as jnp
from jax import lax
from jax.experimental import pallas as pl
from jax.experimental.pallas import tpu as pltpu
from jax.experimental.pallas import tpu_sc as plsc

N = 8192
D = 256
ALPHA = 0.2
RB_H = 512
RB = 192
N_SC = 128
NW = 32
G = 32
L = 16


def _h_kernel(x_ref, w_ref, asrc_ref, adst_ref, hbf_ref, hf_ref,
              av_src_ref, av_dst_ref):
    h = jax.lax.dot_general(
        x_ref[...], w_ref[...],
        dimension_numbers=(((1,), (1,)), ((), ())),
        preferred_element_type=jnp.float32,
    )
    hf_ref[...] = h
    hbf_ref[...] = h.astype(jnp.bfloat16)
    av_src_ref[...] = jnp.sum(h * asrc_ref[...], axis=1, keepdims=True)
    av_dst_ref[...] = jnp.sum(h * adst_ref[...], axis=1, keepdims=True)


def _attn_kernel(adj_ref, asrc_row_ref, adst_col_ref, hbf_ref, out_ref):
    s = asrc_row_ref[...] + adst_col_ref[...]
    s = jnp.where(s > 0, s, ALPHA * s)
    mask = adj_ref[...] > 0
    s = jnp.where(mask, s, 0.0)
    rowmax = jnp.max(s, axis=1, keepdims=True)
    e = jnp.where(mask, jnp.exp(s - rowmax), 0.0)
    inv = 1.0 / (jnp.sum(e, axis=1, keepdims=True) + 1e-8)
    out = jnp.dot(e.astype(jnp.bfloat16), hbf_ref[...],
                  preferred_element_type=jnp.float32) * inv
    out_ref[...] = jnp.where(out > 0, out, jnp.exp(jnp.minimum(out, 0.0)) - 1.0)


def _sc_attn(adj_hbm, asrc_hbm, adst_hbm, h_hbm, out_hbm,
             rowbuf, asrc_v, adst_v, idx_v, w_v, idxc_v, hrows_v, orow_v,
             smem_i, smem_f, sem):
    rows_per_w = N_SC // NW
    wid = lax.axis_index("s") * 2 + lax.axis_index("c")
    row0 = wid * rows_per_w
    zeros = jnp.zeros((L,), jnp.float32)
    zeros_i = jnp.zeros((L,), jnp.int32)
    lane = lax.iota(jnp.int32, L)

    pltpu.sync_copy(asrc_hbm, asrc_v)
    pltpu.sync_copy(adst_hbm, adst_v)

    def row_body(r, carry):
        i = row0 + r
        pltpu.sync_copy(adj_hbm.at[i], rowbuf)

        cb = (i // L) * L
        chunk_d = adst_v[pl.ds(cb, L)]
        sel = jnp.where(lane == (i - cb), chunk_d, 0.0)
        smem_f[2] = jnp.sum(sel)
        adsti = smem_f[2]

        def scan_body(gidx, carry1):
            base = gidx * 4 * L
            v0 = rowbuf[pl.ds(base, L)]
            v1 = rowbuf[pl.ds(base + L, L)]
            v2 = rowbuf[pl.ds(base + 2 * L, L)]
            v3 = rowbuf[pl.ds(base + 3 * L, L)]
            t = jnp.sum(v0 + v1 + v2 + v3)

            def nonempty(c1):
                nnz_c, mrow_c = c1
                for k, vk in enumerate((v0, v1, v2, v3)):
                    m = vk > 0.0
                    mi = m.astype(jnp.int32)
                    s = asrc_v[pl.ds(base + k * L, L)] + adsti
                    s = jnp.where(s > 0, s, ALPHA * s)
                    sm = jnp.where(m, s, -3e38)
                    mrow_c = jnp.maximum(mrow_c, jnp.max(sm))
                    pos = nnz_c + plsc.cumsum(mi) - 1
                    idxc_v[pl.ds(0, L)] = pos
                    pos_rt = idxc_v[pl.ds(0, L)]
                    plsc.store_scatter(idx_v, [pos_rt],
                                       lane + base + k * L, mask=m)
                    plsc.store_scatter(w_v, [pos_rt], s, mask=m)
                    nnz_c = nnz_c + jnp.sum(mi)
                smem_i[0] = nnz_c
                smem_f[0] = mrow_c
                return (nnz_c, mrow_c)

            return lax.cond(t > 0.0, nonempty, lambda c1: c1, carry1)
        smem_i[0] = 0
        smem_f[0] = -3e38
        lax.fori_loop(0, N // (4 * L), scan_body, (0, -3e38))
        nnz = smem_i[0]
        m_row = smem_f[0]
        m_row = jnp.where(nnz < N, jnp.maximum(m_row, 0.0), m_row)

        for k in range(G // L + 1):
            idx_v[pl.ds(nnz + k * L, L)] = zeros_i

        for k in range(G // L):
            idxc_v[pl.ds(k * L, L)] = idx_v[pl.ds(k * L, L)]
        cp0 = pltpu.async_copy(h_hbm.at[idxc_v], hrows_v, sem)

        eb = (nnz + L - 1) // L
        smem_f[1] = 0.0

        def exp_body(k, sum_c):
            base = k * L
            e = jnp.exp(w_v[pl.ds(base, L)] - m_row)
            e = jnp.where((lane + base) < nnz, e, 0.0)
            w_v[pl.ds(base, L)] = e
            sum_n = sum_c + jnp.sum(e)
            smem_f[1] = sum_n
            return sum_n
        lax.fori_loop(0, eb, exp_body, 0.0)
        inv = 1.0 / (jnp.full((L,), smem_f[1], jnp.float32) + 1e-8)
        for k in range(G // L + 1):
            w_v[pl.ds(nnz + k * L, L)] = zeros

        for c in range(D // L):
            orow_v[pl.ds(c * L, L)] = zeros

        def accumulate(base):
            for sub in range(G // L):
                wchunk = w_v[pl.ds(base + sub * L, L)]
                for e in range(L):
                    wb = zeros + wchunk[e]
                    row = sub * L + e
                    for c in range(D // L):
                        plsc.addupdate(orow_v.at[pl.ds(c * L, L)],
                                       wb * hrows_v[row, pl.ds(c * L, L)])

        nch = (nnz + G - 1) // G
        cp0.wait()
        accumulate(0)

        def chunk_body(g, carry2):
            base = g * G
            for k in range(G // L):
                idxc_v[pl.ds(k * L, L)] = idx_v[pl.ds(base + k * L, L)]
            pltpu.async_copy(h_hbm.at[idxc_v], hrows_v, sem).wait()
            accumulate(base)
            return carry2
        lax.fori_loop(1, nch, chunk_body, 0)

        for c in range(D // L):
            v = orow_v[pl.ds(c * L, L)] * inv
            orow_v[pl.ds(c * L, L)] = jnp.where(
                v > 0, v, jnp.exp(jnp.minimum(v, 0.0)) - 1.0)
        pltpu.sync_copy(orow_v, out_hbm.at[i])
        return carry

    lax.fori_loop(0, rows_per_w, row_body, 0)


def kernel(x, adj, W, attn_src, attn_dst):
    hbf, hf, a_src, a_dst = pl.pallas_call(
        _h_kernel,
        grid=(N // RB_H,),
        in_specs=[
            pl.BlockSpec((RB_H, D), lambda i: (i, 0)),
            pl.BlockSpec((D, D), lambda i: (0, 0)),
            pl.BlockSpec((1, D), lambda i: (0, 0)),
            pl.BlockSpec((1, D), lambda i: (0, 0)),
        ],
        out_specs=[
            pl.BlockSpec((RB_H, D), lambda i: (i, 0)),
            pl.BlockSpec((RB_H, D), lambda i: (i, 0)),
            pl.BlockSpec((RB_H, 1), lambda i: (i, 0)),
            pl.BlockSpec((RB_H, 1), lambda i: (i, 0)),
        ],
        out_shape=[
            jax.ShapeDtypeStruct((N, D), jnp.bfloat16),
            jax.ShapeDtypeStruct((N, D), jnp.float32),
            jax.ShapeDtypeStruct((N, 1), jnp.float32),
            jax.ShapeDtypeStruct((N, 1), jnp.float32),
        ],
    )(x, W, attn_src, attn_dst)

    parts = []
    if N_SC < N:
        a_src_row = a_src.reshape(1, N)
        out_tc = pl.pallas_call(
            _attn_kernel,
            grid=((N - N_SC) // RB,),
            in_specs=[
                pl.BlockSpec((RB, N), lambda i: (i + N_SC // RB, 0)),
                pl.BlockSpec((1, N), lambda i: (0, 0)),
                pl.BlockSpec((RB, 1), lambda i: (i + N_SC // RB, 0)),
                pl.BlockSpec((N, D), lambda i: (0, 0)),
            ],
            out_specs=pl.BlockSpec((RB, D), lambda i: (i, 0)),
            out_shape=jax.ShapeDtypeStruct((N - N_SC, D), jnp.float32),
        )(adj, a_src_row, a_dst, hbf)
        parts.append(out_tc)

    if N_SC > 0:
        mesh = plsc.VectorSubcoreMesh(core_axis_name="c", subcore_axis_name="s")
        sc_fn = pl.kernel(
            _sc_attn, mesh=mesh,
            compiler_params=pltpu.CompilerParams(needs_layout_passes=False),
            out_type=jax.ShapeDtypeStruct((N_SC, D), jnp.float32),
            scratch_types=[
                pltpu.VMEM((N,), jnp.float32),
                pltpu.VMEM((N,), jnp.float32),
                pltpu.VMEM((N,), jnp.float32),
                pltpu.VMEM((N + 4 * L,), jnp.int32),
                pltpu.VMEM((N + 4 * L,), jnp.float32),
                pltpu.VMEM((G,), jnp.int32),
                pltpu.VMEM((G, D), jnp.float32),
                pltpu.VMEM((D,), jnp.float32),
                pltpu.SMEM((4,), jnp.int32),
                pltpu.SMEM((4,), jnp.float32),
                pltpu.SemaphoreType.DMA,
            ],
        )
        out_sc = sc_fn(adj, a_src.reshape(N), a_dst.reshape(N), hf)
        parts.insert(0, out_sc)

    return parts[0] if len(parts) == 1 else jnp.concatenate(parts, axis=0)
